# baseline (speedup 1.0000x reference)
import jax
import jax.numpy as jnp
from jax.experimental import pallas as pl
from jax.experimental.pallas import tpu as pltpu

FEAT = 64
NPROT = 8
MOM = 0.9
N_ROWS = 16384
BLK = 2048
NBLK = N_ROWS // BLK
_PREC = jax.lax.Precision.HIGHEST


def _dots_and_inv(x, protos):
    d = jax.lax.dot_general(
        x, protos, (((1,), (1,)), ((), ())),
        preferred_element_type=jnp.float32, precision=_PREC)
    ones = jnp.ones((NPROT, FEAT), jnp.float32)
    s = jax.lax.dot_general(
        x * x, ones, (((1,), (1,)), ((), ())),
        preferred_element_type=jnp.float32, precision=_PREC)
    inv = 1.0 / jnp.maximum(jnp.sqrt(s), 1e-12)
    return d, inv


def _body(src_ref, tgt_ref, protos_ref, out_ref,
          sums_ref, counts_ref, pn_ref, acc_ref):
    t = pl.program_id(0)
    i = pl.program_id(1)

    @pl.when((t == 0) & (i == 0))
    def _init():
        sums_ref[...] = jnp.zeros_like(sums_ref)
        counts_ref[...] = jnp.zeros_like(counts_ref)

    @pl.when(t == 0)
    def _source():
        x = src_ref[...]
        d, inv = _dots_and_inv(x, protos_ref[...])
        dmax = jnp.max(d, axis=1, keepdims=True)
        li = jax.lax.broadcasted_iota(jnp.int32, (BLK, NPROT), 1)
        mi = jnp.min(jnp.where(d == dmax, li, NPROT), axis=1, keepdims=True)
        oh = (li == mi).astype(jnp.float32)
        sums_ref[...] += jax.lax.dot_general(
            oh * inv, x, (((0,), (0,)), ((), ())),
            preferred_element_type=jnp.float32, precision=_PREC)
        counts_ref[...] += jnp.sum(oh, axis=0, keepdims=True)

    @pl.when((t == 1) & (i == 0))
    def _update():
        counts_row = counts_ref[...]
        r = jax.lax.broadcasted_iota(jnp.int32, (NPROT, NPROT), 0)
        c = jax.lax.broadcasted_iota(jnp.int32, (NPROT, NPROT), 1)
        counts_col = jnp.sum(
            jnp.where(r == c, jnp.broadcast_to(counts_row, (NPROT, NPROT)), 0.0),
            axis=1, keepdims=True)
        cm = sums_ref[...] / jnp.maximum(counts_col, 1.0)
        nc = jnp.sqrt(jnp.sum(cm * cm, axis=1, keepdims=True))
        cmn = cm / jnp.maximum(nc, 1e-12)
        protos = protos_ref[...]
        upd = MOM * protos + (1.0 - MOM) * cmn
        upd = jnp.where(counts_col > 0.0, upd, protos)
        nu = jnp.sqrt(jnp.sum(upd * upd, axis=1, keepdims=True))
        pn_ref[...] = upd / jnp.maximum(nu, 1e-12)
        acc_ref[0, 0] = 0.0

    @pl.when(t == 1)
    def _target():
        x = tgt_ref[...]
        d, inv = _dots_and_inv(x, pn_ref[...])
        dmax = jnp.broadcast_to(jnp.max(d, axis=1, keepdims=True), (BLK, NPROT))
        acc_ref[0, 0] += jnp.sum(1.0 - dmax * inv) / NPROT

    @pl.when((t == 1) & (i == NBLK - 1))
    def _finish():
        out_ref[0, 0] = acc_ref[0, 0] / N_ROWS


@jax.jit
def kernel(source_feat, target_feat, prototypes):
    loss = pl.pallas_call(
        _body,
        grid=(2, NBLK),
        in_specs=[
            pl.BlockSpec((BLK, FEAT), lambda t, i: (i * (1 - t) + (NBLK - 1) * t, 0)),
            pl.BlockSpec((BLK, FEAT), lambda t, i: (i * t, 0)),
            pl.BlockSpec((NPROT, FEAT), lambda t, i: (0, 0)),
        ],
        out_specs=pl.BlockSpec((1, 1), lambda t, i: (0, 0), memory_space=pltpu.SMEM),
        out_shape=jax.ShapeDtypeStruct((1, 1), jnp.float32),
        scratch_shapes=[
            pltpu.VMEM((NPROT, FEAT), jnp.float32),
            pltpu.VMEM((1, NPROT), jnp.float32),
            pltpu.VMEM((NPROT, FEAT), jnp.float32),
            pltpu.SMEM((1, 1), jnp.float32),
        ],
    )(source_feat, target_feat, prototypes)
    return loss[0, 0]


# speedup vs baseline: 1.9712x; 1.9712x over previous
import jax
import jax.numpy as jnp
from jax.experimental import pallas as pl
from jax.experimental.pallas import tpu as pltpu

FEAT = 64
NPROT = 8
MOM = 0.9
N_ROWS = 16384
BLK = 2048
NBLK = N_ROWS // BLK
_PREC = jax.lax.Precision.DEFAULT


def _dots_and_inv(x, protos):
    d = jax.lax.dot_general(
        x, protos, (((1,), (1,)), ((), ())),
        preferred_element_type=jnp.float32, precision=_PREC)
    ones = jnp.ones((NPROT, FEAT), jnp.float32)
    s = jax.lax.dot_general(
        x * x, ones, (((1,), (1,)), ((), ())),
        preferred_element_type=jnp.float32, precision=_PREC)
    inv = 1.0 / jnp.maximum(jnp.sqrt(s), 1e-12)
    return d, inv


def _body(src_ref, tgt_ref, protos_ref, out_ref,
          sums_ref, counts_ref, pn_ref, acc_ref):
    t = pl.program_id(0)
    i = pl.program_id(1)

    @pl.when((t == 0) & (i == 0))
    def _init():
        sums_ref[...] = jnp.zeros_like(sums_ref)
        counts_ref[...] = jnp.zeros_like(counts_ref)

    @pl.when(t == 0)
    def _source():
        x = src_ref[...]
        d, inv = _dots_and_inv(x, protos_ref[...])
        dmax = jnp.max(d, axis=1, keepdims=True)
        oh = (d == dmax).astype(jnp.float32)
        sums_ref[...] += jax.lax.dot_general(
            oh * inv, x, (((0,), (0,)), ((), ())),
            preferred_element_type=jnp.float32, precision=_PREC)
        counts_ref[...] += jnp.sum(oh, axis=0, keepdims=True)

    @pl.when((t == 1) & (i == 0))
    def _update():
        counts_row = counts_ref[...]
        r = jax.lax.broadcasted_iota(jnp.int32, (NPROT, NPROT), 0)
        c = jax.lax.broadcasted_iota(jnp.int32, (NPROT, NPROT), 1)
        counts_col = jnp.sum(
            jnp.where(r == c, jnp.broadcast_to(counts_row, (NPROT, NPROT)), 0.0),
            axis=1, keepdims=True)
        cm = sums_ref[...] / jnp.maximum(counts_col, 1.0)
        nc = jnp.sqrt(jnp.sum(cm * cm, axis=1, keepdims=True))
        cmn = cm / jnp.maximum(nc, 1e-12)
        protos = protos_ref[...]
        upd = MOM * protos + (1.0 - MOM) * cmn
        upd = jnp.where(counts_col > 0.0, upd, protos)
        nu = jnp.sqrt(jnp.sum(upd * upd, axis=1, keepdims=True))
        pn_ref[...] = upd / jnp.maximum(nu, 1e-12)
        acc_ref[0, 0] = 0.0

    @pl.when(t == 1)
    def _target():
        x = tgt_ref[...]
        d, inv = _dots_and_inv(x, pn_ref[...])
        dmax = jnp.broadcast_to(jnp.max(d, axis=1, keepdims=True), (BLK, NPROT))
        acc_ref[0, 0] += jnp.sum(1.0 - dmax * inv) / NPROT

    @pl.when((t == 1) & (i == NBLK - 1))
    def _finish():
        out_ref[0, 0] = acc_ref[0, 0] / N_ROWS


@jax.jit
def kernel(source_feat, target_feat, prototypes):
    loss = pl.pallas_call(
        _body,
        grid=(2, NBLK),
        in_specs=[
            pl.BlockSpec((BLK, FEAT), lambda t, i: (i * (1 - t) + (NBLK - 1) * t, 0)),
            pl.BlockSpec((BLK, FEAT), lambda t, i: (i * t, 0)),
            pl.BlockSpec((NPROT, FEAT), lambda t, i: (0, 0)),
        ],
        out_specs=pl.BlockSpec((1, 1), lambda t, i: (0, 0), memory_space=pltpu.SMEM),
        out_shape=jax.ShapeDtypeStruct((1, 1), jnp.float32),
        scratch_shapes=[
            pltpu.VMEM((NPROT, FEAT), jnp.float32),
            pltpu.VMEM((1, NPROT), jnp.float32),
            pltpu.VMEM((NPROT, FEAT), jnp.float32),
            pltpu.SMEM((1, 1), jnp.float32),
        ],
    )(source_feat, target_feat, prototypes)
    return loss[0, 0]


# speedup vs baseline: 2.2021x; 1.1171x over previous
import jax
import jax.numpy as jnp
from jax.experimental import pallas as pl
from jax.experimental.pallas import tpu as pltpu

FEAT = 64
NPROT = 8
MOM = 0.9
N_ROWS = 16384
BLK = 2048
NBLK = N_ROWS // BLK


def _dots_and_inv(x, protos):
    d = jax.lax.dot_general(
        protos, x, (((1,), (1,)), ((), ())),
        preferred_element_type=jnp.float32)
    ones = jnp.ones((1, FEAT), jnp.float32)
    s = jax.lax.dot_general(
        ones, x * x, (((1,), (1,)), ((), ())),
        preferred_element_type=jnp.float32)
    inv = jax.lax.rsqrt(jnp.maximum(s, 1e-24))
    return d, inv


def _body(src_ref, tgt_ref, protos_ref, out_ref,
          sums_ref, counts_ref, pn_ref, acc_ref):
    t = pl.program_id(0)
    i = pl.program_id(1)

    @pl.when((t == 0) & (i == 0))
    def _init():
        sums_ref[...] = jnp.zeros_like(sums_ref)
        counts_ref[...] = jnp.zeros_like(counts_ref)

    @pl.when(t == 0)
    def _source():
        x = src_ref[...]
        d, inv = _dots_and_inv(x, protos_ref[...])
        dmax = jnp.max(d, axis=0, keepdims=True)
        oh = (d == dmax).astype(jnp.float32)
        sums_ref[...] += jax.lax.dot_general(
            oh * inv, x, (((1,), (0,)), ((), ())),
            preferred_element_type=jnp.float32)
        counts_ref[...] += jnp.sum(oh, axis=1, keepdims=True)

    @pl.when((t == 1) & (i == 0))
    def _update():
        counts_col = counts_ref[...]
        cm = sums_ref[...] / jnp.maximum(counts_col, 1.0)
        nc = jnp.sqrt(jnp.sum(cm * cm, axis=1, keepdims=True))
        cmn = cm / jnp.maximum(nc, 1e-12)
        protos = protos_ref[...]
        upd = MOM * protos + (1.0 - MOM) * cmn
        upd = jnp.where(counts_col > 0.0, upd, protos)
        nu = jnp.sqrt(jnp.sum(upd * upd, axis=1, keepdims=True))
        pn_ref[...] = upd / jnp.maximum(nu, 1e-12)
        acc_ref[0, 0] = 0.0

    @pl.when(t == 1)
    def _target():
        x = tgt_ref[...]
        d, inv = _dots_and_inv(x, pn_ref[...])
        dmax = jnp.max(d, axis=0, keepdims=True)
        acc_ref[0, 0] += jnp.sum(1.0 - dmax * inv)

    @pl.when((t == 1) & (i == NBLK - 1))
    def _finish():
        out_ref[0, 0] = acc_ref[0, 0] / N_ROWS


@jax.jit
def kernel(source_feat, target_feat, prototypes):
    loss = pl.pallas_call(
        _body,
        grid=(2, NBLK),
        in_specs=[
            pl.BlockSpec((BLK, FEAT), lambda t, i: (i * (1 - t) + (NBLK - 1) * t, 0)),
            pl.BlockSpec((BLK, FEAT), lambda t, i: (i * t, 0)),
            pl.BlockSpec((NPROT, FEAT), lambda t, i: (0, 0)),
        ],
        out_specs=pl.BlockSpec((1, 1), lambda t, i: (0, 0), memory_space=pltpu.SMEM),
        out_shape=jax.ShapeDtypeStruct((1, 1), jnp.float32),
        scratch_shapes=[
            pltpu.VMEM((NPROT, FEAT), jnp.float32),
            pltpu.VMEM((NPROT, 1), jnp.float32),
            pltpu.VMEM((NPROT, FEAT), jnp.float32),
            pltpu.SMEM((1, 1), jnp.float32),
        ],
    )(source_feat, target_feat, prototypes)
    return loss[0, 0]


# speedup vs baseline: 2.6026x; 1.1819x over previous
import jax
import jax.numpy as jnp
from jax.experimental import pallas as pl
from jax.experimental.pallas import tpu as pltpu

FEAT = 64
NPROT = 8
MOM = 0.9
N_ROWS = 16384
BLK = 4096
NBLK = N_ROWS // BLK


def _dots_and_inv(x, protos):
    d = jax.lax.dot_general(
        protos, x, (((1,), (1,)), ((), ())),
        preferred_element_type=jnp.float32)
    ones = jnp.ones((1, FEAT), jnp.float32)
    s = jax.lax.dot_general(
        ones, x * x, (((1,), (1,)), ((), ())),
        preferred_element_type=jnp.float32)
    inv = jax.lax.rsqrt(jnp.maximum(s, 1e-24))
    return d, inv


def _body(src_ref, tgt_ref, protos_ref, out_ref,
          sums_ref, counts_ref, pn_ref, acc_ref):
    t = pl.program_id(0)
    i = pl.program_id(1)

    @pl.when((t == 0) & (i == 0))
    def _init():
        sums_ref[...] = jnp.zeros_like(sums_ref)
        counts_ref[...] = jnp.zeros_like(counts_ref)

    @pl.when(t == 0)
    def _source():
        x = src_ref[...]
        d, inv = _dots_and_inv(x, protos_ref[...])
        dmax = jnp.max(d, axis=0, keepdims=True)
        oh = (d == dmax).astype(jnp.float32)
        sums_ref[...] += jax.lax.dot_general(
            oh * inv, x, (((1,), (0,)), ((), ())),
            preferred_element_type=jnp.float32)
        counts_ref[...] += jnp.sum(oh, axis=1, keepdims=True)

    @pl.when((t == 1) & (i == 0))
    def _update():
        counts_col = counts_ref[...]
        cm = sums_ref[...] / jnp.maximum(counts_col, 1.0)
        nc = jnp.sqrt(jnp.sum(cm * cm, axis=1, keepdims=True))
        cmn = cm / jnp.maximum(nc, 1e-12)
        protos = protos_ref[...]
        upd = MOM * protos + (1.0 - MOM) * cmn
        upd = jnp.where(counts_col > 0.0, upd, protos)
        nu = jnp.sqrt(jnp.sum(upd * upd, axis=1, keepdims=True))
        pn_ref[...] = upd / jnp.maximum(nu, 1e-12)
        acc_ref[0, 0] = 0.0

    @pl.when(t == 1)
    def _target():
        x = tgt_ref[...]
        d, inv = _dots_and_inv(x, pn_ref[...])
        dmax = jnp.max(d, axis=0, keepdims=True)
        acc_ref[0, 0] += jnp.sum(1.0 - dmax * inv)

    @pl.when((t == 1) & (i == NBLK - 1))
    def _finish():
        out_ref[0, 0] = acc_ref[0, 0] / N_ROWS


@jax.jit
def kernel(source_feat, target_feat, prototypes):
    loss = pl.pallas_call(
        _body,
        grid=(2, NBLK),
        in_specs=[
            pl.BlockSpec((BLK, FEAT), lambda t, i: (i * (1 - t) + (NBLK - 1) * t, 0)),
            pl.BlockSpec((BLK, FEAT), lambda t, i: (i * t, 0)),
            pl.BlockSpec((NPROT, FEAT), lambda t, i: (0, 0)),
        ],
        out_specs=pl.BlockSpec((1, 1), lambda t, i: (0, 0), memory_space=pltpu.SMEM),
        out_shape=jax.ShapeDtypeStruct((1, 1), jnp.float32),
        scratch_shapes=[
            pltpu.VMEM((NPROT, FEAT), jnp.float32),
            pltpu.VMEM((NPROT, 1), jnp.float32),
            pltpu.VMEM((NPROT, FEAT), jnp.float32),
            pltpu.SMEM((1, 1), jnp.float32),
        ],
    )(source_feat, target_feat, prototypes)
    return loss[0, 0]


# speedup vs baseline: 2.7412x; 1.0533x over previous
import jax
import jax.numpy as jnp
from jax.experimental import pallas as pl
from jax.experimental.pallas import tpu as pltpu

FEAT = 64
NPROT = 8
MOM = 0.9
N_ROWS = 16384
BLK = 8192
NBLK = N_ROWS // BLK


def _dots_and_inv(x, protos):
    d = jax.lax.dot_general(
        protos, x, (((1,), (1,)), ((), ())),
        preferred_element_type=jnp.float32)
    ones = jnp.ones((1, FEAT), jnp.float32)
    s = jax.lax.dot_general(
        ones, x * x, (((1,), (1,)), ((), ())),
        preferred_element_type=jnp.float32)
    inv = jax.lax.rsqrt(jnp.maximum(s, 1e-24))
    return d, inv


def _body(src_ref, tgt_ref, protos_ref, out_ref,
          sums_ref, counts_ref, pn_ref, acc_ref):
    t = pl.program_id(0)
    i = pl.program_id(1)

    @pl.when((t == 0) & (i == 0))
    def _init():
        sums_ref[...] = jnp.zeros_like(sums_ref)
        counts_ref[...] = jnp.zeros_like(counts_ref)

    @pl.when(t == 0)
    def _source():
        x = src_ref[...]
        d, inv = _dots_and_inv(x, protos_ref[...])
        dmax = jnp.max(d, axis=0, keepdims=True)
        oh = (d == dmax).astype(jnp.float32)
        sums_ref[...] += jax.lax.dot_general(
            oh * inv, x, (((1,), (0,)), ((), ())),
            preferred_element_type=jnp.float32)
        counts_ref[...] += jnp.sum(oh, axis=1, keepdims=True)

    @pl.when((t == 1) & (i == 0))
    def _update():
        counts_col = counts_ref[...]
        cm = sums_ref[...] / jnp.maximum(counts_col, 1.0)
        nc = jnp.sqrt(jnp.sum(cm * cm, axis=1, keepdims=True))
        cmn = cm / jnp.maximum(nc, 1e-12)
        protos = protos_ref[...]
        upd = MOM * protos + (1.0 - MOM) * cmn
        upd = jnp.where(counts_col > 0.0, upd, protos)
        nu = jnp.sqrt(jnp.sum(upd * upd, axis=1, keepdims=True))
        pn_ref[...] = upd / jnp.maximum(nu, 1e-12)
        acc_ref[0, 0] = 0.0

    @pl.when(t == 1)
    def _target():
        x = tgt_ref[...]
        d, inv = _dots_and_inv(x, pn_ref[...])
        dmax = jnp.max(d, axis=0, keepdims=True)
        acc_ref[0, 0] += jnp.sum(1.0 - dmax * inv)

    @pl.when((t == 1) & (i == NBLK - 1))
    def _finish():
        out_ref[0, 0] = acc_ref[0, 0] / N_ROWS


@jax.jit
def kernel(source_feat, target_feat, prototypes):
    loss = pl.pallas_call(
        _body,
        grid=(2, NBLK),
        in_specs=[
            pl.BlockSpec((BLK, FEAT), lambda t, i: (i * (1 - t) + (NBLK - 1) * t, 0)),
            pl.BlockSpec((BLK, FEAT), lambda t, i: (i * t, 0)),
            pl.BlockSpec((NPROT, FEAT), lambda t, i: (0, 0)),
        ],
        out_specs=pl.BlockSpec((1, 1), lambda t, i: (0, 0), memory_space=pltpu.SMEM),
        out_shape=jax.ShapeDtypeStruct((1, 1), jnp.float32),
        scratch_shapes=[
            pltpu.VMEM((NPROT, FEAT), jnp.float32),
            pltpu.VMEM((NPROT, 1), jnp.float32),
            pltpu.VMEM((NPROT, FEAT), jnp.float32),
            pltpu.SMEM((1, 1), jnp.float32),
        ],
    )(source_feat, target_feat, prototypes)
    return loss[0, 0]
